# baseline (speedup 1.0000x reference)
import jax
import jax.numpy as jnp
import numpy as np
from jax import lax
from jax.experimental import pallas as pl
from jax.experimental.pallas import tpu as pltpu
from jax.experimental.pallas import tpu_sc as plsc

N = 10000
E = 320000
C = 128
H = 4
DH = 32
SCALE = DH ** -0.5

NC = 2
NS = 16
NW = NC * NS
EW = E // NW
SB = 80
CH = 400
NSB = CH // SB
NIT = EW // CH
NROW = EW // SB
G = CH // 16
SCHUNK = H * CH
CP = 80
NIT2 = EW // CP
G2 = CP // 16
SC2 = H * CP
RW = 624
RTAIL = N - NS * RW

_f32 = jnp.float32
_i32 = jnp.int32

_ROWBLK = 1000
_GRID = N // _ROWBLK


def _mesh():
    return plsc.VectorSubcoreMesh(
        core_axis_name="c", subcore_axis_name="s", num_cores=NC,
        num_subcores=NS)


def _proj_body(x_ref, wi_ref, bi_ref, wq_ref, bq_ref, wk_ref, bk_ref,
               wv_ref, bv_ref, xw_ref, q_ref, k_ref, v_ref):
    xw = jnp.dot(x_ref[...], wi_ref[...],
                 preferred_element_type=_f32) + bi_ref[...]
    xw_ref[...] = xw
    q_ref[...] = jnp.dot(xw, wq_ref[...],
                         preferred_element_type=_f32) + bq_ref[...]
    k_ref[...] = jnp.dot(xw, wk_ref[...],
                         preferred_element_type=_f32) + bk_ref[...]
    v_ref[...] = jnp.dot(xw, wv_ref[...],
                         preferred_element_type=_f32) + bv_ref[...]


def _proj(x, Wi, bi, Wq, bq, Wk, bk, Wv, bv):
    row = pl.BlockSpec((_ROWBLK, C), lambda i: (i, 0))
    wsp = pl.BlockSpec((C, C), lambda i: (0, 0))
    bsp = pl.BlockSpec((1, C), lambda i: (0, 0))
    out = jax.ShapeDtypeStruct((N, C), _f32)
    return pl.pallas_call(
        _proj_body,
        grid=(_GRID,),
        in_specs=[row, wsp, bsp, wsp, bsp, wsp, bsp, wsp, bsp],
        out_specs=[row, row, row, row],
        out_shape=[out, out, out, out],
    )(x, Wi, bi, Wq, bq, Wk, bk, Wv, bv)


def _pass1_body(q_hbm, k_hbm, src_hbm, dst_hbm, scores_hbm, maxes_hbm,
                dall, sall, qv, kv, scv, mxv, sem):
    c = lax.axis_index("c")
    s = lax.axis_index("s")
    wid = s * NC + c
    neg = jnp.full((16,), -3.0e38, _f32)
    for h in range(H):
        mxv[pl.ds(h * 16, 16)] = neg
    lane = lax.iota(_i32, 16)

    def it_body(i, _):
        pltpu.sync_copy(dst_hbm.at[wid, i], dall)
        pltpu.sync_copy(src_hbm.at[wid, i], sall)
        cps = []
        for j in range(NSB):
            cps.append(pltpu.async_copy(
                q_hbm.at[dall.at[j]],
                qv.at[pl.ds(j * SB, SB)], sem))
            cps.append(pltpu.async_copy(
                k_hbm.at[sall.at[j]],
                kv.at[pl.ds(j * SB, SB)], sem))
        for cp in cps:
            cp.wait()

        def j_body(j, _):
            jbase = lane + j * SB
            for g in range(G2):
                rows = jbase + g * 16
                for h in range(H):
                    acc = jnp.zeros((16,), _f32)
                    for d in range(DH):
                        col = jnp.full((16,), h * DH + d, _i32)
                        qe = plsc.load_gather(qv, [rows, col])
                        ke = plsc.load_gather(kv, [rows, col])
                        acc = acc + qe * ke
                    acc = acc * SCALE
                    scv[pl.ds(j * SC2 + h * CP + g * 16, 16)] = acc
                    mxv[pl.ds(h * 16, 16)] = jnp.maximum(
                        mxv[pl.ds(h * 16, 16)], acc)
            return 0

        lax.fori_loop(0, NSB, j_body, 0)
        pltpu.sync_copy(scv,
                        scores_hbm.at[pl.ds((wid * NIT + i) * SCHUNK, SCHUNK)])
        return 0

    lax.fori_loop(0, NIT, it_body, 0)
    pltpu.sync_copy(mxv, maxes_hbm.at[pl.ds(wid * H * 16, H * 16)])


def _pass1(q, k, src, dst):
    kfn = pl.kernel(
        _pass1_body,
        out_type=[jax.ShapeDtypeStruct((E * H,), _f32),
                  jax.ShapeDtypeStruct((NW * H * 16,), _f32)],
        mesh=_mesh(),
        scratch_types=[
            pltpu.VMEM((NSB, SB), _i32),
            pltpu.VMEM((NSB, SB), _i32),
            pltpu.VMEM((CH, C), _f32),
            pltpu.VMEM((CH, C), _f32),
            pltpu.VMEM((SCHUNK,), _f32),
            pltpu.VMEM((H * 16,), _f32),
            pltpu.SemaphoreType.DMA,
        ],
        compiler_params=pltpu.CompilerParams(needs_layout_passes=False),
    )
    return kfn(q, k, src, dst)


def _pass2_body(v_hbm, src_hbm, dst_hbm, scores_hbm, maxes_hbm, zeros_hbm,
                outp_hbm, sums_hbm, sall0, dall0, sall1, dall1, gv0, gv1,
                scv0, scv1, ev, smv, mxall, accS, sg0, sg1):
    c = lax.axis_index("c")
    s = lax.axis_index("s")
    wid = s * NC + c

    pltpu.sync_copy(zeros_hbm.at[pl.ds(s * RW, RW)], accS.at[pl.ds(s * RW, RW)])

    @pl.when(s == 0)
    def _zero_tail():
        pltpu.sync_copy(zeros_hbm.at[pl.ds(NS * RW, RTAIL)],
                        accS.at[pl.ds(NS * RW, RTAIL)])

    pltpu.sync_copy(maxes_hbm, mxall)
    zero16 = jnp.zeros((16,), _f32)
    m = []
    for h in range(H):
        mv = jnp.full((16,), -3.0e38, _f32)
        for w in range(NW):
            mv = jnp.maximum(mv, mxall[pl.ds(w * H * 16 + h * 16, 16)])
        m.append(jnp.full((16,), jnp.max(mv), _f32))
        smv[pl.ds(h * 16, 16)] = zero16
    plsc.subcore_barrier()

    def stage(chunk, sall, dall, scv, gv, sg):
        pltpu.sync_copy(dst_hbm.at[wid * NIT2 + chunk], dall)
        pltpu.sync_copy(src_hbm.at[wid * NIT2 + chunk], sall)
        pltpu.sync_copy(scores_hbm.at[pl.ds((wid * NIT2 + chunk) * SC2, SC2)],
                        scv)
        pltpu.make_async_copy(v_hbm.at[sall.at[0]], gv, sg).start()

    def process(sall, dall, scv, gv, sg):
        pltpu.make_async_copy(v_hbm.at[sall.at[0]], gv, sg).wait()

        for g in range(G2):
            for h in range(H):
                e = jnp.exp(scv[pl.ds(h * CP + g * 16, 16)] - m[h])
                smv[pl.ds(h * 16, 16)] = smv[pl.ds(h * 16, 16)] + e
                ev[pl.ds(h * 16, 16)] = e
            for ed in range(16):
                row = g * 16 + ed
                for h in range(H):
                    idx = jnp.full((16,), h * 16 + ed, _i32)
                    eb = plsc.load_gather(ev, [idx])
                    lo = gv[row, pl.ds(h * DH, 16)]
                    hi = gv[row, pl.ds(h * DH + 16, 16)]
                    gv[row, pl.ds(h * DH, 16)] = lo * eb
                    gv[row, pl.ds(h * DH + 16, 16)] = hi * eb
        pltpu.sync_copy(gv, accS.at[dall.at[0]], add=True)

    stage(0, sall0, dall0, scv0, gv0, sg0)
    stage(1, sall1, dall1, scv1, gv1, sg1)

    def it_body(t, _):
        process(sall0, dall0, scv0, gv0, sg0)

        @pl.when(2 * t + 2 < NIT2)
        def _s0():
            stage(2 * t + 2, sall0, dall0, scv0, gv0, sg0)

        process(sall1, dall1, scv1, gv1, sg1)

        @pl.when(2 * t + 3 < NIT2)
        def _s1():
            stage(2 * t + 3, sall1, dall1, scv1, gv1, sg1)

        return 0

    lax.fori_loop(0, (NIT2 - 1) // 2, it_body, 0)
    process(sall0, dall0, scv0, gv0, sg0)
    plsc.subcore_barrier()
    pltpu.sync_copy(accS.at[pl.ds(s * RW, RW)],
                    outp_hbm.at[pl.ds(c * N + s * RW, RW)])

    @pl.when(s == 0)
    def _write_tail():
        pltpu.sync_copy(accS.at[pl.ds(NS * RW, RTAIL)],
                        outp_hbm.at[pl.ds(c * N + NS * RW, RTAIL)])

    pltpu.sync_copy(smv, sums_hbm.at[pl.ds(wid * H * 16, H * 16)])


def _pass2(v, src, dst, scores, maxes, zeros):
    kfn = pl.kernel(
        _pass2_body,
        out_type=[jax.ShapeDtypeStruct((NC * N, C), _f32),
                  jax.ShapeDtypeStruct((NW * H * 16,), _f32)],
        mesh=_mesh(),
        scratch_types=[
            pltpu.VMEM((1, CP), _i32),
            pltpu.VMEM((1, CP), _i32),
            pltpu.VMEM((1, CP), _i32),
            pltpu.VMEM((1, CP), _i32),
            pltpu.VMEM((CP, C), _f32),
            pltpu.VMEM((CP, C), _f32),
            pltpu.VMEM((SC2,), _f32),
            pltpu.VMEM((SC2,), _f32),
            pltpu.VMEM((H * 16,), _f32),
            pltpu.VMEM((H * 16,), _f32),
            pltpu.VMEM((NW * H * 16,), _f32),
            pltpu.VMEM_SHARED((N, C), _f32),
            pltpu.SemaphoreType.DMA,
            pltpu.SemaphoreType.DMA,
        ],
        compiler_params=pltpu.CompilerParams(needs_layout_passes=False),
    )
    return kfn(v, src, dst, scores, maxes, zeros)


def _epi_body(a_ref, b_ref, xw_ref, sums_ref, sel_ref, wo_ref, bo_ref,
              wf1_ref, bf1_ref, wf2_ref, bf2_ref, out_ref):
    inv = 1.0 / jnp.sum(sums_ref[...], axis=1, keepdims=True)
    scale = jnp.sum(sel_ref[...] * inv, axis=0, keepdims=True)
    o = (a_ref[...] + b_ref[...]) * scale
    x1 = xw_ref[...] + jnp.dot(o, wo_ref[...],
                               preferred_element_type=_f32) + bo_ref[...]
    hddn = jnp.maximum(
        jnp.dot(x1, wf1_ref[...], preferred_element_type=_f32)
        + bf1_ref[...], 0.0)
    out_ref[...] = x1 + jnp.dot(hddn, wf2_ref[...],
                                preferred_element_type=_f32) + bf2_ref[...]


def _epi(a, b, xw, sums2, sel, Wo, bo, Wf1, bf1, Wf2, bf2):
    row = pl.BlockSpec((_ROWBLK, C), lambda i: (i, 0))
    return pl.pallas_call(
        _epi_body,
        grid=(_GRID,),
        in_specs=[
            row, row, row,
            pl.BlockSpec((H, NW * 16), lambda i: (0, 0)),
            pl.BlockSpec((H, C), lambda i: (0, 0)),
            pl.BlockSpec((C, C), lambda i: (0, 0)),
            pl.BlockSpec((1, C), lambda i: (0, 0)),
            pl.BlockSpec((C, 2 * C), lambda i: (0, 0)),
            pl.BlockSpec((1, 2 * C), lambda i: (0, 0)),
            pl.BlockSpec((2 * C, C), lambda i: (0, 0)),
            pl.BlockSpec((1, C), lambda i: (0, 0)),
        ],
        out_specs=row,
        out_shape=jax.ShapeDtypeStruct((N, C), _f32),
    )(a, b, xw, sums2, sel, Wo, bo, Wf1, bf1, Wf2, bf2)


_SEL = np.kron(np.eye(H, dtype=np.float32), np.ones((1, DH), np.float32))


def kernel(x, edge_index, Wi, bi, Wq, bq, Wk, bk, Wv, bv, Wo, bo,
           Wf1, bf1, Wf2, bf2):
    src = edge_index[0].reshape(NW, NIT, NSB, SB)
    dst = edge_index[1].reshape(NW, NIT, NSB, SB)
    src2 = edge_index[0].reshape(NW * NIT2, 1, CP)
    dst2 = edge_index[1].reshape(NW * NIT2, 1, CP)
    xw, q, k, v = _proj(x, Wi, bi.reshape(1, C), Wq, bq.reshape(1, C),
                        Wk, bk.reshape(1, C), Wv, bv.reshape(1, C))
    scores, maxes = _pass1(q, k, src, dst)
    zeros = jnp.zeros((N, C), _f32)
    outp, sums = _pass2(v, src2, dst2, scores, maxes, zeros)
    sums2 = sums.reshape(NW, H, 16).transpose(1, 0, 2).reshape(H, NW * 16)
    sel = jnp.asarray(_SEL)
    return _epi(outp[:N], outp[N:], xw, sums2, sel, Wo, bo.reshape(1, C),
                Wf1, bf1.reshape(1, 2 * C), Wf2, bf2.reshape(1, C))


# speedup vs baseline: 1.7264x; 1.7264x over previous
import jax
import jax.numpy as jnp
import numpy as np
from jax import lax
from jax.experimental import pallas as pl
from jax.experimental.pallas import tpu as pltpu
from jax.experimental.pallas import tpu_sc as plsc

N = 10000
E = 320000
C = 128
H = 4
DH = 32
SCALE = DH ** -0.5

NC = 2
NS = 16
NW = NC * NS
EW = E // NW
SB = 80
CH = 400
NSB = CH // SB
NIT = EW // CH
NROW = EW // SB
G = CH // 16
SCHUNK = H * CH
CP = 80
NIT2 = EW // CP
G2 = CP // 16
SC2 = H * CP
RW = 624
RTAIL = N - NS * RW

_f32 = jnp.float32
_i32 = jnp.int32

_ROWBLK = 1000
_GRID = N // _ROWBLK


def _mesh():
    return plsc.VectorSubcoreMesh(
        core_axis_name="c", subcore_axis_name="s", num_cores=NC,
        num_subcores=NS)


def _proj_body(x_ref, wi_ref, bi_ref, wq_ref, bq_ref, wk_ref, bk_ref,
               wv_ref, bv_ref, xw_ref, q_ref, k_ref, v_ref):
    xw = jnp.dot(x_ref[...], wi_ref[...],
                 preferred_element_type=_f32) + bi_ref[...]
    xw_ref[...] = xw
    q_ref[...] = jnp.dot(xw, wq_ref[...],
                         preferred_element_type=_f32) + bq_ref[...]
    k_ref[...] = jnp.dot(xw, wk_ref[...],
                         preferred_element_type=_f32) + bk_ref[...]
    v_ref[...] = jnp.dot(xw, wv_ref[...],
                         preferred_element_type=_f32) + bv_ref[...]


def _proj(x, Wi, bi, Wq, bq, Wk, bk, Wv, bv):
    row = pl.BlockSpec((_ROWBLK, C), lambda i: (i, 0))
    wsp = pl.BlockSpec((C, C), lambda i: (0, 0))
    bsp = pl.BlockSpec((1, C), lambda i: (0, 0))
    out = jax.ShapeDtypeStruct((N, C), _f32)
    return pl.pallas_call(
        _proj_body,
        grid=(_GRID,),
        in_specs=[row, wsp, bsp, wsp, bsp, wsp, bsp, wsp, bsp],
        out_specs=[row, row, row, row],
        out_shape=[out, out, out, out],
    )(x, Wi, bi, Wq, bq, Wk, bk, Wv, bv)


def _pass1_body(q_hbm, k_hbm, src_hbm, dst_hbm, scores_hbm, maxes_hbm,
                dall, sall, qv, kv, scv, mxv, sem):
    c = lax.axis_index("c")
    s = lax.axis_index("s")
    wid = s * NC + c
    neg = jnp.full((16,), -3.0e38, _f32)
    for h in range(H):
        mxv[pl.ds(h * 16, 16)] = neg
    lane = lax.iota(_i32, 16)
    mask15 = lane == jnp.full((16,), 15, _i32)

    def it_body(i, _):
        pltpu.sync_copy(dst_hbm.at[wid, i], dall)
        pltpu.sync_copy(src_hbm.at[wid, i], sall)
        cps = []
        for j in range(NSB):
            cps.append(pltpu.async_copy(
                q_hbm.at[dall.at[j]],
                qv.at[pl.ds(j * SB, SB)], sem))
            cps.append(pltpu.async_copy(
                k_hbm.at[sall.at[j]],
                kv.at[pl.ds(j * SB, SB)], sem))
        for cp in cps:
            cp.wait()

        def e_body(e, _):
            sub = e // SB
            pos = e - sub * SB
            off = sub * SC2 + pos
            for h in range(H):
                p0 = (qv[e, pl.ds(h * DH, 16)] * kv[e, pl.ds(h * DH, 16)]
                      + qv[e, pl.ds(h * DH + 16, 16)]
                      * kv[e, pl.ds(h * DH + 16, 16)])
                cs = plsc.cumsum(p0 * SCALE)
                plsc.store_scatter(
                    scv, [jnp.full((16,), off + h * CP, _i32)], cs,
                    mask=mask15)
            return 0

        lax.fori_loop(0, CH, e_body, 0)

        def mx_body(t, _):
            for h in range(H):
                sc16 = scv[pl.ds(t * 16 + h * CP, 16)]
                mxv[pl.ds(h * 16, 16)] = jnp.maximum(
                    mxv[pl.ds(h * 16, 16)], sc16)
            return 0

        def mxo_body(j, _):
            def mxi_body(g, _):
                for h in range(H):
                    sc16 = scv[pl.ds(j * SC2 + h * CP + g * 16, 16)]
                    mxv[pl.ds(h * 16, 16)] = jnp.maximum(
                        mxv[pl.ds(h * 16, 16)], sc16)
                return 0
            lax.fori_loop(0, G2, mxi_body, 0)
            return 0

        lax.fori_loop(0, NSB, mxo_body, 0)
        pltpu.sync_copy(scv,
                        scores_hbm.at[pl.ds((wid * NIT + i) * SCHUNK, SCHUNK)])
        return 0

    lax.fori_loop(0, NIT, it_body, 0)
    pltpu.sync_copy(mxv, maxes_hbm.at[pl.ds(wid * H * 16, H * 16)])


def _pass1(q, k, src, dst):
    kfn = pl.kernel(
        _pass1_body,
        out_type=[jax.ShapeDtypeStruct((E * H,), _f32),
                  jax.ShapeDtypeStruct((NW * H * 16,), _f32)],
        mesh=_mesh(),
        scratch_types=[
            pltpu.VMEM((NSB, SB), _i32),
            pltpu.VMEM((NSB, SB), _i32),
            pltpu.VMEM((CH, C), _f32),
            pltpu.VMEM((CH, C), _f32),
            pltpu.VMEM((SCHUNK,), _f32),
            pltpu.VMEM((H * 16,), _f32),
            pltpu.SemaphoreType.DMA,
        ],
        compiler_params=pltpu.CompilerParams(needs_layout_passes=False),
    )
    return kfn(q, k, src, dst)


def _pass2_body(v_hbm, src_hbm, dst_hbm, scores_hbm, maxes_hbm, zeros_hbm,
                outp_hbm, sums_hbm, sall0, dall0, sall1, dall1, gv0, gv1,
                scv0, scv1, ev, smv, mxall, accS, sg0, sg1):
    c = lax.axis_index("c")
    s = lax.axis_index("s")
    wid = s * NC + c

    pltpu.sync_copy(zeros_hbm.at[pl.ds(s * RW, RW)], accS.at[pl.ds(s * RW, RW)])

    @pl.when(s == 0)
    def _zero_tail():
        pltpu.sync_copy(zeros_hbm.at[pl.ds(NS * RW, RTAIL)],
                        accS.at[pl.ds(NS * RW, RTAIL)])

    pltpu.sync_copy(maxes_hbm, mxall)
    zero16 = jnp.zeros((16,), _f32)
    m = []
    for h in range(H):
        mv = jnp.full((16,), -3.0e38, _f32)
        for w in range(NW):
            mv = jnp.maximum(mv, mxall[pl.ds(w * H * 16 + h * 16, 16)])
        m.append(jnp.full((16,), jnp.max(mv), _f32))
        smv[pl.ds(h * 16, 16)] = zero16
    plsc.subcore_barrier()

    def stage(chunk, sall, dall, scv, gv, sg):
        pltpu.sync_copy(dst_hbm.at[wid * NIT2 + chunk], dall)
        pltpu.sync_copy(src_hbm.at[wid * NIT2 + chunk], sall)
        pltpu.sync_copy(scores_hbm.at[pl.ds((wid * NIT2 + chunk) * SC2, SC2)],
                        scv)
        pltpu.make_async_copy(v_hbm.at[sall.at[0]], gv, sg).start()

    def process(sall, dall, scv, gv, sg):
        pltpu.make_async_copy(v_hbm.at[sall.at[0]], gv, sg).wait()

        def g_body(g, _):
            for h in range(H):
                e = jnp.exp(scv[pl.ds(h * CP + g * 16, 16)] - m[h])
                smv[pl.ds(h * 16, 16)] = smv[pl.ds(h * 16, 16)] + e
                ev[pl.ds(h * 16, 16)] = e
            for ed in range(16):
                row = g * 16 + ed
                for h in range(H):
                    idx = jnp.full((16,), h * 16 + ed, _i32)
                    eb = plsc.load_gather(ev, [idx])
                    lo = gv[row, pl.ds(h * DH, 16)]
                    hi = gv[row, pl.ds(h * DH + 16, 16)]
                    gv[row, pl.ds(h * DH, 16)] = lo * eb
                    gv[row, pl.ds(h * DH + 16, 16)] = hi * eb
            return 0

        lax.fori_loop(0, G2, g_body, 0)
        pltpu.sync_copy(gv, accS.at[dall.at[0]], add=True)

    stage(0, sall0, dall0, scv0, gv0, sg0)
    stage(1, sall1, dall1, scv1, gv1, sg1)

    def it_body(t, _):
        process(sall0, dall0, scv0, gv0, sg0)

        @pl.when(2 * t + 2 < NIT2)
        def _s0():
            stage(2 * t + 2, sall0, dall0, scv0, gv0, sg0)

        process(sall1, dall1, scv1, gv1, sg1)

        @pl.when(2 * t + 3 < NIT2)
        def _s1():
            stage(2 * t + 3, sall1, dall1, scv1, gv1, sg1)

        return 0

    lax.fori_loop(0, (NIT2 - 1) // 2, it_body, 0)
    process(sall0, dall0, scv0, gv0, sg0)
    plsc.subcore_barrier()
    pltpu.sync_copy(accS.at[pl.ds(s * RW, RW)],
                    outp_hbm.at[pl.ds(c * N + s * RW, RW)])

    @pl.when(s == 0)
    def _write_tail():
        pltpu.sync_copy(accS.at[pl.ds(NS * RW, RTAIL)],
                        outp_hbm.at[pl.ds(c * N + NS * RW, RTAIL)])

    pltpu.sync_copy(smv, sums_hbm.at[pl.ds(wid * H * 16, H * 16)])


def _pass2(v, src, dst, scores, maxes, zeros):
    kfn = pl.kernel(
        _pass2_body,
        out_type=[jax.ShapeDtypeStruct((NC * N, C), _f32),
                  jax.ShapeDtypeStruct((NW * H * 16,), _f32)],
        mesh=_mesh(),
        scratch_types=[
            pltpu.VMEM((1, CP), _i32),
            pltpu.VMEM((1, CP), _i32),
            pltpu.VMEM((1, CP), _i32),
            pltpu.VMEM((1, CP), _i32),
            pltpu.VMEM((CP, C), _f32),
            pltpu.VMEM((CP, C), _f32),
            pltpu.VMEM((SC2,), _f32),
            pltpu.VMEM((SC2,), _f32),
            pltpu.VMEM((H * 16,), _f32),
            pltpu.VMEM((H * 16,), _f32),
            pltpu.VMEM((NW * H * 16,), _f32),
            pltpu.VMEM_SHARED((N, C), _f32),
            pltpu.SemaphoreType.DMA,
            pltpu.SemaphoreType.DMA,
        ],
        compiler_params=pltpu.CompilerParams(needs_layout_passes=False),
    )
    return kfn(v, src, dst, scores, maxes, zeros)


def _epi_body(a_ref, b_ref, xw_ref, sums_ref, sel_ref, wo_ref, bo_ref,
              wf1_ref, bf1_ref, wf2_ref, bf2_ref, out_ref):
    inv = 1.0 / jnp.sum(sums_ref[...], axis=1, keepdims=True)
    scale = jnp.sum(sel_ref[...] * inv, axis=0, keepdims=True)
    o = (a_ref[...] + b_ref[...]) * scale
    x1 = xw_ref[...] + jnp.dot(o, wo_ref[...],
                               preferred_element_type=_f32) + bo_ref[...]
    hddn = jnp.maximum(
        jnp.dot(x1, wf1_ref[...], preferred_element_type=_f32)
        + bf1_ref[...], 0.0)
    out_ref[...] = x1 + jnp.dot(hddn, wf2_ref[...],
                                preferred_element_type=_f32) + bf2_ref[...]


def _epi(a, b, xw, sums2, sel, Wo, bo, Wf1, bf1, Wf2, bf2):
    row = pl.BlockSpec((_ROWBLK, C), lambda i: (i, 0))
    return pl.pallas_call(
        _epi_body,
        grid=(_GRID,),
        in_specs=[
            row, row, row,
            pl.BlockSpec((H, NW * 16), lambda i: (0, 0)),
            pl.BlockSpec((H, C), lambda i: (0, 0)),
            pl.BlockSpec((C, C), lambda i: (0, 0)),
            pl.BlockSpec((1, C), lambda i: (0, 0)),
            pl.BlockSpec((C, 2 * C), lambda i: (0, 0)),
            pl.BlockSpec((1, 2 * C), lambda i: (0, 0)),
            pl.BlockSpec((2 * C, C), lambda i: (0, 0)),
            pl.BlockSpec((1, C), lambda i: (0, 0)),
        ],
        out_specs=row,
        out_shape=jax.ShapeDtypeStruct((N, C), _f32),
    )(a, b, xw, sums2, sel, Wo, bo, Wf1, bf1, Wf2, bf2)


_SEL = np.kron(np.eye(H, dtype=np.float32), np.ones((1, DH), np.float32))


def kernel(x, edge_index, Wi, bi, Wq, bq, Wk, bk, Wv, bv, Wo, bo,
           Wf1, bf1, Wf2, bf2):
    src = edge_index[0].reshape(NW, NIT, NSB, SB)
    dst = edge_index[1].reshape(NW, NIT, NSB, SB)
    src2 = edge_index[0].reshape(NW * NIT2, 1, CP)
    dst2 = edge_index[1].reshape(NW * NIT2, 1, CP)
    xw, q, k, v = _proj(x, Wi, bi.reshape(1, C), Wq, bq.reshape(1, C),
                        Wk, bk.reshape(1, C), Wv, bv.reshape(1, C))
    scores, maxes = _pass1(q, k, src, dst)
    zeros = jnp.zeros((N, C), _f32)
    outp, sums = _pass2(v, src2, dst2, scores, maxes, zeros)
    sums2 = sums.reshape(NW, H, 16).transpose(1, 0, 2).reshape(H, NW * 16)
    sel = jnp.asarray(_SEL)
    return _epi(outp[:N], outp[N:], xw, sums2, sel, Wo, bo.reshape(1, C),
                Wf1, bf1.reshape(1, 2 * C), Wf2, bf2.reshape(1, C))


# speedup vs baseline: 1.7285x; 1.0013x over previous
import jax
import jax.numpy as jnp
import numpy as np
from jax import lax
from jax.experimental import pallas as pl
from jax.experimental.pallas import tpu as pltpu
from jax.experimental.pallas import tpu_sc as plsc

N = 10000
E = 320000
C = 128
H = 4
DH = 32
SCALE = DH ** -0.5

NC = 2
NS = 16
NW = NC * NS
EW = E // NW
SB = 80
CH = 400
NSB = CH // SB
NIT = EW // CH
NROW = EW // SB
G = CH // 16
SCHUNK = H * CH
CP = 80
NIT2 = EW // CP
G2 = CP // 16
SC2 = H * CP
RW = 624
RTAIL = N - NS * RW

_f32 = jnp.float32
_i32 = jnp.int32

_ROWBLK = 1000
_GRID = N // _ROWBLK


def _mesh():
    return plsc.VectorSubcoreMesh(
        core_axis_name="c", subcore_axis_name="s", num_cores=NC,
        num_subcores=NS)


def _proj_body(x_ref, wi_ref, bi_ref, wq_ref, bq_ref, wk_ref, bk_ref,
               wv_ref, bv_ref, xw_ref, q_ref, k_ref, v_ref):
    xw = jnp.dot(x_ref[...], wi_ref[...],
                 preferred_element_type=_f32) + bi_ref[...]
    xw_ref[...] = xw
    q_ref[...] = jnp.dot(xw, wq_ref[...],
                         preferred_element_type=_f32) + bq_ref[...]
    k_ref[...] = jnp.dot(xw, wk_ref[...],
                         preferred_element_type=_f32) + bk_ref[...]
    v_ref[...] = jnp.dot(xw, wv_ref[...],
                         preferred_element_type=_f32) + bv_ref[...]


def _proj(x, Wi, bi, Wq, bq, Wk, bk, Wv, bv):
    row = pl.BlockSpec((_ROWBLK, C), lambda i: (i, 0))
    wsp = pl.BlockSpec((C, C), lambda i: (0, 0))
    bsp = pl.BlockSpec((1, C), lambda i: (0, 0))
    out = jax.ShapeDtypeStruct((N, C), _f32)
    return pl.pallas_call(
        _proj_body,
        grid=(_GRID,),
        in_specs=[row, wsp, bsp, wsp, bsp, wsp, bsp, wsp, bsp],
        out_specs=[row, row, row, row],
        out_shape=[out, out, out, out],
    )(x, Wi, bi, Wq, bq, Wk, bk, Wv, bv)


def _pass1_body(q_hbm, k_hbm, src_hbm, dst_hbm, scores_hbm, maxes_hbm,
                dall, sall, qv, kv, scv, mxv, sem):
    c = lax.axis_index("c")
    s = lax.axis_index("s")
    wid = s * NC + c
    neg = jnp.full((16,), -3.0e38, _f32)
    for h in range(H):
        mxv[pl.ds(h * 16, 16)] = neg
    lane = lax.iota(_i32, 16)
    mask15 = lane == jnp.full((16,), 15, _i32)

    def it_body(i, _):
        pltpu.sync_copy(dst_hbm.at[wid, i], dall)
        pltpu.sync_copy(src_hbm.at[wid, i], sall)
        cps = []
        for j in range(NSB):
            cps.append(pltpu.async_copy(
                q_hbm.at[dall.at[j]],
                qv.at[pl.ds(j * SB, SB)], sem))
            cps.append(pltpu.async_copy(
                k_hbm.at[sall.at[j]],
                kv.at[pl.ds(j * SB, SB)], sem))
        for cp in cps:
            cp.wait()

        def e_body(e, _):
            sub = e // SB
            pos = e - sub * SB
            off = sub * SC2 + pos
            for h in range(H):
                p0 = (qv[e, pl.ds(h * DH, 16)] * kv[e, pl.ds(h * DH, 16)]
                      + qv[e, pl.ds(h * DH + 16, 16)]
                      * kv[e, pl.ds(h * DH + 16, 16)])
                cs = plsc.cumsum(p0 * SCALE)
                plsc.store_scatter(
                    scv, [jnp.full((16,), off + h * CP, _i32)], cs,
                    mask=mask15)
            return 0

        lax.fori_loop(0, CH, e_body, 0)

        def mxo_body(j, _):
            def mxi_body(g, _):
                for h in range(H):
                    sc16 = scv[pl.ds(j * SC2 + h * CP + g * 16, 16)]
                    mxv[pl.ds(h * 16, 16)] = jnp.maximum(
                        mxv[pl.ds(h * 16, 16)], sc16)
                return 0
            lax.fori_loop(0, G2, mxi_body, 0)
            return 0

        lax.fori_loop(0, NSB, mxo_body, 0)
        pltpu.sync_copy(scv,
                        scores_hbm.at[pl.ds((wid * NIT + i) * SCHUNK, SCHUNK)])
        return 0

    lax.fori_loop(0, NIT, it_body, 0)
    pltpu.sync_copy(mxv, maxes_hbm.at[pl.ds(wid * H * 16, H * 16)])


def _pass1(q, k, src, dst):
    kfn = pl.kernel(
        _pass1_body,
        out_type=[jax.ShapeDtypeStruct((E * H,), _f32),
                  jax.ShapeDtypeStruct((NW * H * 16,), _f32)],
        mesh=_mesh(),
        scratch_types=[
            pltpu.VMEM((NSB, SB), _i32),
            pltpu.VMEM((NSB, SB), _i32),
            pltpu.VMEM((CH, C), _f32),
            pltpu.VMEM((CH, C), _f32),
            pltpu.VMEM((SCHUNK,), _f32),
            pltpu.VMEM((H * 16,), _f32),
            pltpu.SemaphoreType.DMA,
        ],
        compiler_params=pltpu.CompilerParams(needs_layout_passes=False),
    )
    return kfn(q, k, src, dst)


def _pass2_body(v_hbm, src_hbm, dst_hbm, scores_hbm, maxes_hbm, zeros_hbm,
                outp_hbm, sums_hbm, sall0, dall0, sall1, dall1, gv0, gv1,
                scv0, scv1, ev, smv, mxall, accS, sg0, sg1):
    c = lax.axis_index("c")
    s = lax.axis_index("s")
    wid = s * NC + c

    pltpu.sync_copy(zeros_hbm.at[pl.ds(s * RW, RW)], accS.at[pl.ds(s * RW, RW)])

    @pl.when(s == 0)
    def _zero_tail():
        pltpu.sync_copy(zeros_hbm.at[pl.ds(NS * RW, RTAIL)],
                        accS.at[pl.ds(NS * RW, RTAIL)])

    pltpu.sync_copy(maxes_hbm, mxall)
    zero16 = jnp.zeros((16,), _f32)
    m = []
    for h in range(H):
        mv = jnp.full((16,), -3.0e38, _f32)
        for w in range(NW):
            mv = jnp.maximum(mv, mxall[pl.ds(w * H * 16 + h * 16, 16)])
        m.append(jnp.full((16,), jnp.max(mv), _f32))
        smv[pl.ds(h * 16, 16)] = zero16
    plsc.subcore_barrier()

    def stage(chunk, sall, dall, scv, gv, sg):
        pltpu.sync_copy(dst_hbm.at[wid * NIT2 + chunk], dall)
        pltpu.sync_copy(src_hbm.at[wid * NIT2 + chunk], sall)
        pltpu.sync_copy(scores_hbm.at[pl.ds((wid * NIT2 + chunk) * SC2, SC2)],
                        scv)
        pltpu.make_async_copy(v_hbm.at[sall.at[0]], gv, sg).start()

    def process(sall, dall, scv, gv, sg):
        pltpu.make_async_copy(v_hbm.at[sall.at[0]], gv, sg).wait()

        def g_body(g, _):
            for h in range(H):
                e = jnp.exp(scv[pl.ds(h * CP + g * 16, 16)] - m[h])
                smv[pl.ds(h * 16, 16)] = smv[pl.ds(h * 16, 16)] + e
                ev[pl.ds(h * 16, 16)] = e
            for ed in range(16):
                row = g * 16 + ed
                for h in range(H):
                    idx = jnp.full((16,), h * 16 + ed, _i32)
                    eb = plsc.load_gather(ev, [idx])
                    lo = gv[row, pl.ds(h * DH, 16)]
                    hi = gv[row, pl.ds(h * DH + 16, 16)]
                    gv[row, pl.ds(h * DH, 16)] = lo * eb
                    gv[row, pl.ds(h * DH + 16, 16)] = hi * eb
            return 0

        lax.fori_loop(0, G2, g_body, 0)
        pltpu.sync_copy(gv, accS.at[dall.at[0]], add=True)

    stage(0, sall0, dall0, scv0, gv0, sg0)
    stage(1, sall1, dall1, scv1, gv1, sg1)

    def it_body(t, _):
        process(sall0, dall0, scv0, gv0, sg0)

        @pl.when(2 * t + 2 < NIT2)
        def _s0():
            stage(2 * t + 2, sall0, dall0, scv0, gv0, sg0)

        process(sall1, dall1, scv1, gv1, sg1)

        @pl.when(2 * t + 3 < NIT2)
        def _s1():
            stage(2 * t + 3, sall1, dall1, scv1, gv1, sg1)

        return 0

    lax.fori_loop(0, (NIT2 - 1) // 2, it_body, 0)
    process(sall0, dall0, scv0, gv0, sg0)
    plsc.subcore_barrier()
    pltpu.sync_copy(accS.at[pl.ds(s * RW, RW)],
                    outp_hbm.at[pl.ds(c * N + s * RW, RW)])

    @pl.when(s == 0)
    def _write_tail():
        pltpu.sync_copy(accS.at[pl.ds(NS * RW, RTAIL)],
                        outp_hbm.at[pl.ds(c * N + NS * RW, RTAIL)])

    pltpu.sync_copy(smv, sums_hbm.at[pl.ds(wid * H * 16, H * 16)])


def _pass2(v, src, dst, scores, maxes, zeros):
    kfn = pl.kernel(
        _pass2_body,
        out_type=[jax.ShapeDtypeStruct((NC * N, C), _f32),
                  jax.ShapeDtypeStruct((NW * H * 16,), _f32)],
        mesh=_mesh(),
        scratch_types=[
            pltpu.VMEM((1, CP), _i32),
            pltpu.VMEM((1, CP), _i32),
            pltpu.VMEM((1, CP), _i32),
            pltpu.VMEM((1, CP), _i32),
            pltpu.VMEM((CP, C), _f32),
            pltpu.VMEM((CP, C), _f32),
            pltpu.VMEM((SC2,), _f32),
            pltpu.VMEM((SC2,), _f32),
            pltpu.VMEM((H * 16,), _f32),
            pltpu.VMEM((H * 16,), _f32),
            pltpu.VMEM((NW * H * 16,), _f32),
            pltpu.VMEM_SHARED((N, C), _f32),
            pltpu.SemaphoreType.DMA,
            pltpu.SemaphoreType.DMA,
        ],
        compiler_params=pltpu.CompilerParams(needs_layout_passes=False),
    )
    return kfn(v, src, dst, scores, maxes, zeros)


def _epi_body(a_ref, b_ref, xw_ref, sums_ref, sel_ref, wo_ref, bo_ref,
              wf1_ref, bf1_ref, wf2_ref, bf2_ref, out_ref):
    inv = 1.0 / jnp.sum(sums_ref[...], axis=1, keepdims=True)
    scale = jnp.sum(sel_ref[...] * inv, axis=0, keepdims=True)
    o = (a_ref[...] + b_ref[...]) * scale
    x1 = xw_ref[...] + jnp.dot(o, wo_ref[...],
                               preferred_element_type=_f32) + bo_ref[...]
    hddn = jnp.maximum(
        jnp.dot(x1, wf1_ref[...], preferred_element_type=_f32)
        + bf1_ref[...], 0.0)
    out_ref[...] = x1 + jnp.dot(hddn, wf2_ref[...],
                                preferred_element_type=_f32) + bf2_ref[...]


def _epi(a, b, xw, sums2, sel, Wo, bo, Wf1, bf1, Wf2, bf2):
    row = pl.BlockSpec((_ROWBLK, C), lambda i: (i, 0))
    return pl.pallas_call(
        _epi_body,
        grid=(_GRID,),
        in_specs=[
            row, row, row,
            pl.BlockSpec((H, NW * 16), lambda i: (0, 0)),
            pl.BlockSpec((H, C), lambda i: (0, 0)),
            pl.BlockSpec((C, C), lambda i: (0, 0)),
            pl.BlockSpec((1, C), lambda i: (0, 0)),
            pl.BlockSpec((C, 2 * C), lambda i: (0, 0)),
            pl.BlockSpec((1, 2 * C), lambda i: (0, 0)),
            pl.BlockSpec((2 * C, C), lambda i: (0, 0)),
            pl.BlockSpec((1, C), lambda i: (0, 0)),
        ],
        out_specs=row,
        out_shape=jax.ShapeDtypeStruct((N, C), _f32),
    )(a, b, xw, sums2, sel, Wo, bo, Wf1, bf1, Wf2, bf2)


_SEL = np.kron(np.eye(H, dtype=np.float32), np.ones((1, DH), np.float32))


def kernel(x, edge_index, Wi, bi, Wq, bq, Wk, bk, Wv, bv, Wo, bo,
           Wf1, bf1, Wf2, bf2):
    src = edge_index[0].reshape(NW, NIT, NSB, SB)
    dst = edge_index[1].reshape(NW, NIT, NSB, SB)
    src2 = edge_index[0].reshape(NW * NIT2, 1, CP)
    dst2 = edge_index[1].reshape(NW * NIT2, 1, CP)
    xw, q, k, v = _proj(x, Wi, bi.reshape(1, C), Wq, bq.reshape(1, C),
                        Wk, bk.reshape(1, C), Wv, bv.reshape(1, C))
    scores, maxes = _pass1(q, k, src, dst)
    zeros = jnp.zeros((N, C), _f32)
    outp, sums = _pass2(v, src2, dst2, scores, maxes, zeros)
    sums2 = sums.reshape(NW, H, 16).transpose(1, 0, 2).reshape(H, NW * 16)
    sel = jnp.asarray(_SEL)
    return _epi(outp[:N], outp[N:], xw, sums2, sel, Wo, bo.reshape(1, C),
                Wf1, bf1.reshape(1, 2 * C), Wf2, bf2.reshape(1, C))
